# Initial kernel scaffold; baseline (speedup 1.0000x reference)
import jax
import jax.numpy as jnp
from jax.experimental import pallas as pl


def kernel(input, embedding):
    raise NotImplementedError("write your pallas kernel here")


# speedup vs baseline: 1.8300x; 1.8300x over previous
import functools

import jax
import jax.numpy as jnp
from jax import lax
from jax.experimental import pallas as pl
from jax.experimental.pallas import tpu as pltpu
from jax.experimental.pallas import tpu_sc as plsc

EMBED_DIM = 64
CHUNK = 128
K = 4
GROUP = CHUNK * K


@functools.cache
def _build(n_flat: int, vocab: int):
    info = plsc.get_sparse_core_info()
    nw = info.num_cores * info.num_subcores
    assert n_flat % (nw * GROUP) == 0
    b_per_w = n_flat // nw
    n_groups = b_per_w // GROUP

    mesh = plsc.VectorSubcoreMesh(core_axis_name="c", subcore_axis_name="s")

    @functools.partial(
        pl.kernel,
        out_type=jax.ShapeDtypeStruct((n_flat, EMBED_DIM), jnp.float32),
        mesh=mesh,
        compiler_params=pltpu.CompilerParams(use_tc_tiling_on_sc=False),
        scratch_types=[
            pltpu.VMEM((b_per_w,), jnp.int32),
            pltpu.VMEM((GROUP, EMBED_DIM), jnp.float32),
            pltpu.SemaphoreType.DMA,
        ],
    )
    def gather_kernel(table_hbm, idx_hbm, out_hbm, idx_v, rows_v, sem):
        wid = lax.axis_index("s") * info.num_cores + lax.axis_index("c")
        base = wid * b_per_w
        pltpu.sync_copy(idx_hbm.at[pl.ds(base, b_per_w)], idx_v)

        def group_body(g, carry):
            off = pl.multiple_of(g * GROUP, GROUP)
            copies = []
            for j in range(K):
                copies.append(pltpu.async_copy(
                    table_hbm.at[idx_v.at[pl.ds(off + j * CHUNK, CHUNK)]],
                    rows_v.at[pl.ds(j * CHUNK, CHUNK)],
                    sem,
                ))
            for c in copies:
                c.wait()
            pltpu.sync_copy(rows_v, out_hbm.at[pl.ds(base + off, GROUP)])
            return carry

        lax.fori_loop(0, n_groups, group_body, 0)

    return gather_kernel


def kernel(input, embedding):
    b, h = input.shape
    idx_flat = input.reshape(-1).astype(jnp.int32)
    out_flat = _build(idx_flat.shape[0], embedding.shape[0])(embedding, idx_flat)
    return out_flat.reshape(b, h, EMBED_DIM)


# speedup vs baseline: 1.8627x; 1.0179x over previous
import functools

import jax
import jax.numpy as jnp
from jax import lax
from jax.experimental import pallas as pl
from jax.experimental.pallas import tpu as pltpu
from jax.experimental.pallas import tpu_sc as plsc

EMBED_DIM = 64
CHUNK = 128
K = 4
GROUP = CHUNK * K


@functools.cache
def _build(n_flat: int):
    info = plsc.get_sparse_core_info()
    nw = info.num_cores * info.num_subcores
    assert n_flat % (nw * 2 * GROUP) == 0
    b_per_w = n_flat // nw
    n_groups = b_per_w // GROUP

    mesh = plsc.VectorSubcoreMesh(core_axis_name="c", subcore_axis_name="s")

    @functools.partial(
        pl.kernel,
        out_type=jax.ShapeDtypeStruct((n_flat, EMBED_DIM), jnp.float32),
        mesh=mesh,
        compiler_params=pltpu.CompilerParams(use_tc_tiling_on_sc=False),
        scratch_types=[
            pltpu.VMEM((b_per_w,), jnp.int32),
            pltpu.VMEM((GROUP, EMBED_DIM), jnp.float32),
            pltpu.VMEM((GROUP, EMBED_DIM), jnp.float32),
            pltpu.SemaphoreType.DMA,
            pltpu.SemaphoreType.DMA,
            pltpu.SemaphoreType.DMA,
            pltpu.SemaphoreType.DMA,
        ],
    )
    def gather_kernel(table_hbm, idx_hbm, out_hbm, idx_v,
                      buf_a, buf_b, gs_a, gs_b, os_a, os_b):
        wid = lax.axis_index("s") * info.num_cores + lax.axis_index("c")
        base = wid * b_per_w
        pltpu.sync_copy(idx_hbm.at[pl.ds(base, b_per_w)], idx_v)

        def fire_gather(g, buf, sem):
            off = pl.multiple_of(g * GROUP, GROUP)
            for j in range(K):
                pltpu.async_copy(
                    table_hbm.at[idx_v.at[pl.ds(off + j * CHUNK, CHUNK)]],
                    buf.at[pl.ds(j * CHUNK, CHUNK)],
                    sem,
                )

        def fire_out(g, buf, sem):
            off = pl.multiple_of(g * GROUP, GROUP)
            pltpu.async_copy(buf, out_hbm.at[pl.ds(base + off, GROUP)], sem)

        def drain_gather(buf, sem):
            pltpu.make_async_copy(out_hbm.at[pl.ds(base, GROUP)], buf, sem).wait()

        def drain_out(buf, sem):
            pltpu.make_async_copy(buf, out_hbm.at[pl.ds(base, GROUP)], sem).wait()

        fire_gather(0, buf_a, gs_a)
        fire_gather(1, buf_b, gs_b)

        def pair_body(t, carry):
            g = t * 2
            drain_gather(buf_a, gs_a)
            fire_out(g, buf_a, os_a)
            drain_gather(buf_b, gs_b)
            fire_out(g + 1, buf_b, os_b)
            drain_out(buf_a, os_a)
            fire_gather(g + 2, buf_a, gs_a)
            drain_out(buf_b, os_b)
            fire_gather(g + 3, buf_b, gs_b)
            return carry

        lax.fori_loop(0, n_groups // 2 - 1, pair_body, 0)

        g_last = n_groups - 2
        drain_gather(buf_a, gs_a)
        fire_out(g_last, buf_a, os_a)
        drain_gather(buf_b, gs_b)
        fire_out(g_last + 1, buf_b, os_b)
        drain_out(buf_a, os_a)
        drain_out(buf_b, os_b)

    return gather_kernel


def kernel(input, embedding):
    b, h = input.shape
    idx_flat = input.reshape(-1).astype(jnp.int32)
    out_flat = _build(idx_flat.shape[0])(embedding, idx_flat)
    return out_flat.reshape(b, h, EMBED_DIM)
